# baseline (speedup 1.0000x reference)
import functools

import jax
import jax.numpy as jnp
from jax import lax
from jax.experimental import pallas as pl
from jax.experimental.pallas import tpu as pltpu
from jax.experimental.pallas import tpu_sc as plsc

N_EVENTS = 6000
N_OBJECTS = 4000
N_NODES = 10000
N_HEDGES = 2000
N_INC = 320000
D = 128

NC = 2
NS = 16
NW = NC * NS

NP = 10240
CP = 2048
RB = 640
SW = NP // (NC * RB)
ROWS_SC = RB * SW
IPT = N_INC // NS
CS = 4000
RN = RB * CP
TILE_ELS = RN // NS

_mesh = lambda: plsc.VectorSubcoreMesh(
    core_axis_name="c", subcore_axis_name="s", num_cores=NC, num_subcores=NS)

_sc_params = lambda: pltpu.CompilerParams(needs_layout_passes=False)


ZB = 8192
DUMP = 2048


def _build_body(flat, out, flat_v, off_v, ones_v, zero_v, accum):
  c = lax.axis_index("c")
  s = lax.axis_index("s")
  pltpu.sync_copy(flat.at[s], flat_v)
  zeros16 = jnp.zeros((16,), jnp.float32)
  ones16 = jnp.ones((16,), jnp.float32)

  def of(i, _):
    ones_v[pl.ds(i * 16, 16)] = ones16
    return 0
  lax.fori_loop(0, CS // 16, of, 0)

  def zf(i, _):
    zero_v[pl.ds(i * 16, 16)] = zeros16
    return 0
  lax.fori_loop(0, ZB // 16, zf, 0)

  def sweep(k, _):
    base = (c * ROWS_SC + k * RB) * CP

    def z(i, _):
      pltpu.sync_copy(zero_v, accum.at[pl.ds(s * TILE_ELS + i * ZB, ZB)])
      return 0
    lax.fori_loop(0, TILE_ELS // ZB, z, 0)
    plsc.subcore_barrier()

    def chunk(q, _):
      def off_i(i, _):
        fv = flat_v[pl.ds(q * CS + i * 16, 16)]
        off = fv - base
        inb = (off >= 0) & (off < RN)
        off_v[pl.ds(i * 16, 16)] = jnp.where(
            inb, off, RN + (fv & (DUMP - 1)))
        return 0
      lax.fori_loop(0, CS // 16, off_i, 0)
      pltpu.sync_copy(ones_v, accum.at[off_v], add=True)
      return 0
    lax.fori_loop(0, IPT // CS, chunk, 0)
    plsc.subcore_barrier()

    row0 = c * ROWS_SC + k * RB
    pltpu.sync_copy(accum.at[pl.ds(s * TILE_ELS, TILE_ELS)],
                    out.at[pl.ds(row0 * CP + s * TILE_ELS, TILE_ELS)])
    plsc.subcore_barrier()
    return 0
  lax.fori_loop(0, SW, sweep, 0)


@jax.jit
def _sc_build(flat):
  return pl.kernel(
      _build_body,
      out_type=jax.ShapeDtypeStruct((NP * CP,), jnp.float32),
      mesh=_mesh(),
      compiler_params=_sc_params(),
      scratch_types=[
          pltpu.VMEM((IPT,), jnp.int32),
          pltpu.VMEM((CS,), jnp.int32),
          pltpu.VMEM((CS,), jnp.float32),
          pltpu.VMEM((ZB,), jnp.float32),
          pltpu.VMEM_SHARED((RN + DUMP,), jnp.float32),
      ],
  )(flat)


def _gather_body(table, idx, out, idx_v, rows_v, sem):
  c = lax.axis_index("c")
  s = lax.axis_index("s")
  wid = c * NS + s
  bpw = 2048 // NW
  base = wid * bpw
  pltpu.sync_copy(idx.at[pl.ds(base, bpw)], idx_v)
  pltpu.async_copy(table.at[idx_v], rows_v, sem).wait()
  pltpu.sync_copy(rows_v, out.at[pl.ds(base, bpw)])


@jax.jit
def _sc_gather(table, idx):
  bpw = 2048 // NW
  return pl.kernel(
      _gather_body,
      out_type=jax.ShapeDtypeStruct((2048, D), jnp.float32),
      mesh=_mesh(),
      compiler_params=_sc_params(),
      scratch_types=[
          pltpu.VMEM((bpw,), jnp.int32),
          pltpu.VMEM((bpw, D), jnp.float32),
          pltpu.SemaphoreType.DMA,
      ],
  )(table, idx)


def _proj_body(x_ref, w_ref, b_ref, g_ref, be_ref, o_ref):
  y = jnp.dot(x_ref[...], w_ref[...], preferred_element_type=jnp.float32)
  y = y + b_ref[...]
  y = jnp.where(y >= 0, y, 0.2 * y)
  m = jnp.mean(y, axis=-1, keepdims=True)
  v = jnp.mean((y - m) ** 2, axis=-1, keepdims=True)
  o_ref[...] = (y - m) / jnp.sqrt(v + 1e-5) * g_ref[...] + be_ref[...]


@jax.jit
def _proj(x, w, b, g, be):
  n = x.shape[0]
  rb = 1000
  grid = n // rb
  return pl.pallas_call(
      _proj_body,
      grid=(grid,),
      in_specs=[
          pl.BlockSpec((rb, D), lambda i: (i, 0)),
          pl.BlockSpec((D, D), lambda i: (0, 0)),
          pl.BlockSpec((1, D), lambda i: (0, 0)),
          pl.BlockSpec((1, D), lambda i: (0, 0)),
          pl.BlockSpec((1, D), lambda i: (0, 0)),
      ],
      out_specs=pl.BlockSpec((rb, D), lambda i: (i, 0)),
      out_shape=jax.ShapeDtypeStruct((n, D), jnp.float32),
  )(x, w, b.reshape(1, D), g.reshape(1, D), be.reshape(1, D))


def _cast_body(c_ref, cb_ref, nc_ref, ec_ref, ecacc):
  i = pl.program_id(0)
  c = c_ref[...]
  cb_ref[...] = c.astype(jnp.bfloat16)
  rs = jnp.sum(c, axis=1, keepdims=True)
  nc_ref[...] = 1.0 / jnp.maximum(rs, 1.0)

  @pl.when(i == 0)
  def _():
    ecacc[...] = jnp.zeros_like(ecacc)

  ecacc[...] += jnp.sum(c, axis=0, keepdims=True)

  @pl.when(i == pl.num_programs(0) - 1)
  def _():
    ec_ref[...] = 1.0 / jnp.maximum(ecacc[...], 1.0)


@jax.jit
def _cast(cmat):
  rb = 1024
  grid = NP // rb
  return pl.pallas_call(
      _cast_body,
      grid=(grid,),
      in_specs=[pl.BlockSpec((rb, CP), lambda i: (i, 0))],
      out_specs=[
          pl.BlockSpec((rb, CP), lambda i: (i, 0)),
          pl.BlockSpec((rb, 1), lambda i: (i, 0)),
          pl.BlockSpec((1, CP), lambda i: (0, 0)),
      ],
      out_shape=[
          jax.ShapeDtypeStruct((NP, CP), jnp.bfloat16),
          jax.ShapeDtypeStruct((NP, 1), jnp.float32),
          jax.ShapeDtypeStruct((1, CP), jnp.float32),
      ],
      scratch_shapes=[pltpu.VMEM((1, CP), jnp.float32)],
  )(cmat)


def _edge_body(c_ref, x_ref, ec_ref, o_ref, acc_ref):
  i = pl.program_id(0)

  @pl.when(i == 0)
  def _():
    acc_ref[...] = jnp.zeros_like(acc_ref)

  acc_ref[...] += lax.dot_general(
      c_ref[...], x_ref[...], (((0,), (0,)), ((), ())),
      preferred_element_type=jnp.float32)

  @pl.when(i == pl.num_programs(0) - 1)
  def _():
    o_ref[...] = acc_ref[...] * ec_ref[...]


@jax.jit
def _edge_mm(cb, x, ecr):
  rb = 1024
  grid = NP // rb
  return pl.pallas_call(
      _edge_body,
      grid=(grid,),
      in_specs=[
          pl.BlockSpec((rb, CP), lambda i: (i, 0)),
          pl.BlockSpec((rb, D), lambda i: (i, 0)),
          pl.BlockSpec((CP, 1), lambda i: (0, 0)),
      ],
      out_specs=pl.BlockSpec((CP, D), lambda i: (0, 0)),
      out_shape=jax.ShapeDtypeStruct((CP, D), jnp.float32),
      scratch_shapes=[pltpu.VMEM((CP, D), jnp.float32)],
  )(cb, x, ecr)


def _node_body(c_ref, ea_ref, nc_ref, x_ref, w_ref, b_ref, o_ref):
  z = jnp.dot(c_ref[...], ea_ref[...], preferred_element_type=jnp.float32)
  nf = z * nc_ref[...]
  y = jnp.dot(nf, w_ref[...], preferred_element_type=jnp.float32) + b_ref[...]
  o_ref[...] = jnp.maximum(y, 0.0) + x_ref[...]


@jax.jit
def _node_mm(cb, ea, ncr, xres, w, b):
  rb = 1024
  grid = NP // rb
  return pl.pallas_call(
      _node_body,
      grid=(grid,),
      in_specs=[
          pl.BlockSpec((rb, CP), lambda i: (i, 0)),
          pl.BlockSpec((CP, D), lambda i: (0, 0)),
          pl.BlockSpec((rb, 1), lambda i: (i, 0)),
          pl.BlockSpec((rb, D), lambda i: (i, 0)),
          pl.BlockSpec((D, D), lambda i: (0, 0)),
          pl.BlockSpec((1, D), lambda i: (0, 0)),
      ],
      out_specs=pl.BlockSpec((rb, D), lambda i: (i, 0)),
      out_shape=jax.ShapeDtypeStruct((NP, D), jnp.float32),
  )(cb, ea, ncr, xres, w, b.reshape(1, D))


def _fusion_body(ev_ref, ob_ref, w1_ref, w2_ref, b_ref, o_ref):
  ev = ev_ref[...]
  ob = ob_ref[...]
  z = (jnp.dot(ob, w1_ref[...], preferred_element_type=jnp.float32)
       + jnp.dot(ev, w2_ref[...], preferred_element_type=jnp.float32)
       + b_ref[...])
  g = jax.nn.sigmoid(z)
  o_ref[...] = g * ob + (1.0 - g) * ev


@jax.jit
def _fusion(ev, ob, w1, w2, b):
  n = ev.shape[0]
  return pl.pallas_call(
      _fusion_body,
      grid=(1,),
      in_specs=[
          pl.BlockSpec((n, D), lambda i: (0, 0)),
          pl.BlockSpec((n, D), lambda i: (0, 0)),
          pl.BlockSpec((D, D), lambda i: (0, 0)),
          pl.BlockSpec((D, D), lambda i: (0, 0)),
          pl.BlockSpec((1, D), lambda i: (0, 0)),
      ],
      out_specs=pl.BlockSpec((n, D), lambda i: (0, 0)),
      out_shape=jax.ShapeDtypeStruct((n, D), jnp.float32),
  )(ev, ob, w1, w2, b.reshape(1, D))


def kernel(object_X, event_X, W_ev, b_ev, g_ev, be_ev, W_ob, b_ob, g_ob, be_ob,
           W1, b1, W2, b2, Wg, bg, node_idx, hedge_idx, main_object, event_sel):
  flat = (node_idx * CP + hedge_idx).reshape(NS, IPT)
  c1d = _sc_build(flat)
  cmat = c1d.reshape(NP, CP)
  cb, ncr, ecr = _cast(cmat)
  ecc = ecr.reshape(CP, 1)

  ev = _proj(event_X, W_ev, b_ev, g_ev, be_ev)
  ob = _proj(object_X, W_ob, b_ob, g_ob, be_ob)
  X = jnp.concatenate([ev, ob, jnp.zeros((NP - N_NODES, D), jnp.float32)],
                      axis=0)

  ef1 = _edge_mm(cb, X.astype(jnp.bfloat16), ecc)
  H1 = _node_mm(cb, ef1.astype(jnp.bfloat16), ncr, X, W1, b1)
  ef2 = _edge_mm(cb, H1.astype(jnp.bfloat16), ecc)
  H2 = _node_mm(cb, ef2.astype(jnp.bfloat16), ncr, H1, W2, b2)

  sel = jnp.concatenate([event_sel, main_object + N_EVENTS], axis=0)
  rows = _sc_gather(H2, sel)
  return _fusion(rows[:1024], rows[1024:], Wg[:D], Wg[D:], bg)


# speedup vs baseline: 1.0358x; 1.0358x over previous
import functools

import jax
import jax.numpy as jnp
from jax import lax
from jax.experimental import pallas as pl
from jax.experimental.pallas import tpu as pltpu
from jax.experimental.pallas import tpu_sc as plsc

N_EVENTS = 6000
N_OBJECTS = 4000
N_NODES = 10000
N_HEDGES = 2000
N_INC = 320000
D = 128

NC = 2
NS = 16
NW = NC * NS

NP = 10240
CP = 2048
HR = NP // 2
RB = 640
SW = HR // (NC * RB)
ROWS_SC = RB * SW
IPT = N_INC // NS
CS = 4000
RN = RB * CP
TILE_ELS = RN // NS

_mesh = lambda: plsc.VectorSubcoreMesh(
    core_axis_name="c", subcore_axis_name="s", num_cores=NC, num_subcores=NS)

_sc_params = lambda: pltpu.CompilerParams(needs_layout_passes=False)


ZB = 8192
DUMP = 2048


def _build_body(h, flat, out, flat_v, off_v, ones_v, zero_v, accum):
  c = lax.axis_index("c")
  s = lax.axis_index("s")
  pltpu.sync_copy(flat.at[s], flat_v)
  zeros16 = jnp.zeros((16,), jnp.float32)
  ones16 = jnp.ones((16,), jnp.float32)

  def of(i, _):
    ones_v[pl.ds(i * 16, 16)] = ones16
    return 0
  lax.fori_loop(0, CS // 16, of, 0)

  def zf(i, _):
    zero_v[pl.ds(i * 16, 16)] = zeros16
    return 0
  lax.fori_loop(0, ZB // 16, zf, 0)

  def sweep(k, _):
    base = (h * HR + c * ROWS_SC + k * RB) * CP

    def z(i, _):
      pltpu.sync_copy(zero_v, accum.at[pl.ds(s * TILE_ELS + i * ZB, ZB)])
      return 0
    lax.fori_loop(0, TILE_ELS // ZB, z, 0)
    plsc.subcore_barrier()

    def chunk(q, _):
      def off_i(i, _):
        fv = flat_v[pl.ds(q * CS + i * 16, 16)]
        off = fv - base
        inb = (off >= 0) & (off < RN)
        off_v[pl.ds(i * 16, 16)] = jnp.where(
            inb, off, RN + (fv & (DUMP - 1)))
        return 0
      lax.fori_loop(0, CS // 16, off_i, 0)
      pltpu.sync_copy(ones_v, accum.at[off_v], add=True)
      return 0
    lax.fori_loop(0, IPT // CS, chunk, 0)
    plsc.subcore_barrier()

    row0 = c * ROWS_SC + k * RB
    pltpu.sync_copy(accum.at[pl.ds(s * TILE_ELS, TILE_ELS)],
                    out.at[pl.ds(row0 * CP + s * TILE_ELS, TILE_ELS)])
    plsc.subcore_barrier()
    return 0
  lax.fori_loop(0, SW, sweep, 0)


@functools.partial(jax.jit, static_argnums=0)
def _sc_build(h, flat):
  return pl.kernel(
      functools.partial(_build_body, h),
      out_type=jax.ShapeDtypeStruct((HR * CP,), jnp.float32),
      mesh=_mesh(),
      compiler_params=_sc_params(),
      scratch_types=[
          pltpu.VMEM((IPT,), jnp.int32),
          pltpu.VMEM((CS,), jnp.int32),
          pltpu.VMEM((CS,), jnp.float32),
          pltpu.VMEM((ZB,), jnp.float32),
          pltpu.VMEM_SHARED((RN + DUMP,), jnp.float32),
      ],
  )(flat)


def _gather_body(table, idx, out, idx_v, rows_v, sem):
  c = lax.axis_index("c")
  s = lax.axis_index("s")
  wid = c * NS + s
  bpw = 2048 // NW
  base = wid * bpw
  pltpu.sync_copy(idx.at[pl.ds(base, bpw)], idx_v)
  pltpu.async_copy(table.at[idx_v], rows_v, sem).wait()
  pltpu.sync_copy(rows_v, out.at[pl.ds(base, bpw)])


@jax.jit
def _sc_gather(table, idx):
  bpw = 2048 // NW
  return pl.kernel(
      _gather_body,
      out_type=jax.ShapeDtypeStruct((2048, D), jnp.float32),
      mesh=_mesh(),
      compiler_params=_sc_params(),
      scratch_types=[
          pltpu.VMEM((bpw,), jnp.int32),
          pltpu.VMEM((bpw, D), jnp.float32),
          pltpu.SemaphoreType.DMA,
      ],
  )(table, idx)


def _proj_body(x_ref, w_ref, b_ref, g_ref, be_ref, o_ref):
  y = jnp.dot(x_ref[...], w_ref[...], preferred_element_type=jnp.float32)
  y = y + b_ref[...]
  y = jnp.where(y >= 0, y, 0.2 * y)
  m = jnp.mean(y, axis=-1, keepdims=True)
  v = jnp.mean((y - m) ** 2, axis=-1, keepdims=True)
  o_ref[...] = (y - m) / jnp.sqrt(v + 1e-5) * g_ref[...] + be_ref[...]


@jax.jit
def _proj(x, w, b, g, be):
  n = x.shape[0]
  rb = 1000
  grid = n // rb
  return pl.pallas_call(
      _proj_body,
      grid=(grid,),
      in_specs=[
          pl.BlockSpec((rb, D), lambda i: (i, 0)),
          pl.BlockSpec((D, D), lambda i: (0, 0)),
          pl.BlockSpec((1, D), lambda i: (0, 0)),
          pl.BlockSpec((1, D), lambda i: (0, 0)),
          pl.BlockSpec((1, D), lambda i: (0, 0)),
      ],
      out_specs=pl.BlockSpec((rb, D), lambda i: (i, 0)),
      out_shape=jax.ShapeDtypeStruct((n, D), jnp.float32),
  )(x, w, b.reshape(1, D), g.reshape(1, D), be.reshape(1, D))


def _cast_body(c_ref, cb_ref, nc_ref, ec_ref, ecacc):
  i = pl.program_id(0)
  c = c_ref[...]
  cb_ref[...] = c.astype(jnp.bfloat16)
  rs = jnp.sum(c, axis=1, keepdims=True)
  nc_ref[...] = 1.0 / jnp.maximum(rs, 1.0)

  @pl.when(i == 0)
  def _():
    ecacc[...] = jnp.zeros_like(ecacc)

  ecacc[...] += jnp.sum(c, axis=0, keepdims=True)

  @pl.when(i == pl.num_programs(0) - 1)
  def _():
    ec_ref[...] = ecacc[...]


@jax.jit
def _cast(cmat):
  rb = 1024
  grid = HR // rb
  return pl.pallas_call(
      _cast_body,
      grid=(grid,),
      in_specs=[pl.BlockSpec((rb, CP), lambda i: (i, 0))],
      out_specs=[
          pl.BlockSpec((rb, CP), lambda i: (i, 0)),
          pl.BlockSpec((rb, 1), lambda i: (i, 0)),
          pl.BlockSpec((1, CP), lambda i: (0, 0)),
      ],
      out_shape=[
          jax.ShapeDtypeStruct((HR, CP), jnp.bfloat16),
          jax.ShapeDtypeStruct((HR, 1), jnp.float32),
          jax.ShapeDtypeStruct((1, CP), jnp.float32),
      ],
      scratch_shapes=[pltpu.VMEM((1, CP), jnp.float32)],
  )(cmat)


def _edge_body(c_ref, x_ref, o_ref, acc_ref):
  i = pl.program_id(0)

  @pl.when(i == 0)
  def _():
    acc_ref[...] = jnp.zeros_like(acc_ref)

  acc_ref[...] += lax.dot_general(
      c_ref[...], x_ref[...], (((0,), (0,)), ((), ())),
      preferred_element_type=jnp.float32)

  @pl.when(i == pl.num_programs(0) - 1)
  def _():
    o_ref[...] = acc_ref[...]


@jax.jit
def _edge_mm(cb, x):
  rb = 1024
  grid = HR // rb
  return pl.pallas_call(
      _edge_body,
      grid=(grid,),
      in_specs=[
          pl.BlockSpec((rb, CP), lambda i: (i, 0)),
          pl.BlockSpec((rb, D), lambda i: (i, 0)),
      ],
      out_specs=pl.BlockSpec((CP, D), lambda i: (0, 0)),
      out_shape=jax.ShapeDtypeStruct((CP, D), jnp.float32),
      scratch_shapes=[pltpu.VMEM((CP, D), jnp.float32)],
  )(cb, x)


def _combine_body(p1_ref, p2_ref, e1_ref, e2_ref, o_ref):
  scale = 1.0 / jnp.maximum(e1_ref[...] + e2_ref[...], 1.0)
  ef = (p1_ref[...] + p2_ref[...]) * scale
  o_ref[...] = ef.astype(jnp.bfloat16)


@jax.jit
def _combine(p1, p2, e1, e2):
  return pl.pallas_call(
      _combine_body,
      grid=(1,),
      in_specs=[
          pl.BlockSpec((CP, D), lambda i: (0, 0)),
          pl.BlockSpec((CP, D), lambda i: (0, 0)),
          pl.BlockSpec((CP, 1), lambda i: (0, 0)),
          pl.BlockSpec((CP, 1), lambda i: (0, 0)),
      ],
      out_specs=pl.BlockSpec((CP, D), lambda i: (0, 0)),
      out_shape=jax.ShapeDtypeStruct((CP, D), jnp.bfloat16),
  )(p1, p2, e1.reshape(CP, 1), e2.reshape(CP, 1))


def _node_body(c_ref, ea_ref, nc_ref, x_ref, w_ref, b_ref, o_ref):
  z = jnp.dot(c_ref[...], ea_ref[...], preferred_element_type=jnp.float32)
  nf = z * nc_ref[...]
  y = jnp.dot(nf, w_ref[...], preferred_element_type=jnp.float32) + b_ref[...]
  o_ref[...] = jnp.maximum(y, 0.0) + x_ref[...]


@jax.jit
def _node_mm(cb, ea, ncr, xres, w, b):
  rb = 1024
  grid = HR // rb
  return pl.pallas_call(
      _node_body,
      grid=(grid,),
      in_specs=[
          pl.BlockSpec((rb, CP), lambda i: (i, 0)),
          pl.BlockSpec((CP, D), lambda i: (0, 0)),
          pl.BlockSpec((rb, 1), lambda i: (i, 0)),
          pl.BlockSpec((rb, D), lambda i: (i, 0)),
          pl.BlockSpec((D, D), lambda i: (0, 0)),
          pl.BlockSpec((1, D), lambda i: (0, 0)),
      ],
      out_specs=pl.BlockSpec((rb, D), lambda i: (i, 0)),
      out_shape=jax.ShapeDtypeStruct((HR, D), jnp.float32),
  )(cb, ea, ncr, xres, w, b.reshape(1, D))


def _fusion_body(ev_ref, ob_ref, w1_ref, w2_ref, b_ref, o_ref):
  ev = ev_ref[...]
  ob = ob_ref[...]
  z = (jnp.dot(ob, w1_ref[...], preferred_element_type=jnp.float32)
       + jnp.dot(ev, w2_ref[...], preferred_element_type=jnp.float32)
       + b_ref[...])
  g = jax.nn.sigmoid(z)
  o_ref[...] = g * ob + (1.0 - g) * ev


@jax.jit
def _fusion(ev, ob, w1, w2, b):
  n = ev.shape[0]
  return pl.pallas_call(
      _fusion_body,
      grid=(1,),
      in_specs=[
          pl.BlockSpec((n, D), lambda i: (0, 0)),
          pl.BlockSpec((n, D), lambda i: (0, 0)),
          pl.BlockSpec((D, D), lambda i: (0, 0)),
          pl.BlockSpec((D, D), lambda i: (0, 0)),
          pl.BlockSpec((1, D), lambda i: (0, 0)),
      ],
      out_specs=pl.BlockSpec((n, D), lambda i: (0, 0)),
      out_shape=jax.ShapeDtypeStruct((n, D), jnp.float32),
  )(ev, ob, w1, w2, b.reshape(1, D))


def kernel(object_X, event_X, W_ev, b_ev, g_ev, be_ev, W_ob, b_ob, g_ob, be_ob,
           W1, b1, W2, b2, Wg, bg, node_idx, hedge_idx, main_object, event_sel):
  flat = (node_idx * CP + hedge_idx).reshape(NS, IPT)
  ca = _sc_build(0, flat).reshape(HR, CP)
  cb2_raw = _sc_build(1, flat).reshape(HR, CP)
  cba, ncra, eca = _cast(ca)
  cbb, ncrb, ecb = _cast(cb2_raw)

  ev = _proj(event_X, W_ev, b_ev, g_ev, be_ev)
  ob = _proj(object_X, W_ob, b_ob, g_ob, be_ob)
  X = jnp.concatenate([ev, ob, jnp.zeros((NP - N_NODES, D), jnp.float32)],
                      axis=0)
  Xa, Xb = X[:HR], X[HR:]

  p1a = _edge_mm(cba, Xa.astype(jnp.bfloat16))
  p1b = _edge_mm(cbb, Xb.astype(jnp.bfloat16))
  ef1 = _combine(p1a, p1b, eca, ecb)
  H1a = _node_mm(cba, ef1, ncra, Xa, W1, b1)
  H1b = _node_mm(cbb, ef1, ncrb, Xb, W1, b1)
  p2a = _edge_mm(cba, H1a.astype(jnp.bfloat16))
  p2b = _edge_mm(cbb, H1b.astype(jnp.bfloat16))
  ef2 = _combine(p2a, p2b, eca, ecb)
  H2a = _node_mm(cba, ef2, ncra, H1a, W2, b2)
  H2b = _node_mm(cbb, ef2, ncrb, H1b, W2, b2)
  H2 = jnp.concatenate([H2a, H2b], axis=0)

  sel = jnp.concatenate([event_sel, main_object + N_EVENTS], axis=0)
  rows = _sc_gather(H2, sel)
  return _fusion(rows[:1024], rows[1024:], Wg[:D], Wg[D:], bg)
